# baseline (speedup 1.0000x reference)
import functools

import jax
import jax.numpy as jnp
from jax import lax
from jax.experimental import pallas as pl
from jax.experimental.pallas import tpu as pltpu
from jax.experimental.pallas import tpu_sc as plsc

_NNEG = 20
_PAD = 448
_CH = 112
_NCH = 4
_NW = 32


def _sc_scores(B, C, D, iword, idx_all, emb_i, emb_o):
    bpw = B // _NW
    mesh = plsc.VectorSubcoreMesh(core_axis_name="c", subcore_axis_name="s")
    ngrp = _CH // 16

    @functools.partial(
        pl.kernel,
        mesh=mesh,
        compiler_params=pltpu.CompilerParams(needs_layout_passes=False),
        out_type=jax.ShapeDtypeStruct((B, _PAD), jnp.float32),
        scratch_types=[
            pltpu.VMEM((bpw,), jnp.int32),
            pltpu.VMEM((bpw, D), jnp.float32),
            pltpu.VMEM((_NCH, _CH), jnp.int32),
        ] + [pltpu.VMEM((_CH, D), jnp.float32) for _ in range(_NCH)] + [
            pltpu.VMEM((_PAD,), jnp.float32),
            pltpu.SemaphoreType.DMA,
            pltpu.SemaphoreType.DMA,
        ],
    )
    def body(iword_hbm, idx_hbm, emb_i_hbm, emb_o_hbm, out_hbm,
             iw_idx, ivecs, idx_row, rows0, rows1, rows2, rows3,
             scores_row, isem, gsem):
        rows = [rows0, rows1, rows2, rows3]
        wid = lax.axis_index("s") * 2 + lax.axis_index("c")
        base = wid * bpw
        pltpu.sync_copy(iword_hbm.at[pl.ds(base, bpw)], iw_idx)
        pltpu.async_copy(emb_i_hbm.at[iw_idx], ivecs, isem).wait()

        def batch_body(bl, carry):
            b = base + bl
            blvec = jnp.full((16,), bl, jnp.int32)
            pltpu.sync_copy(idx_hbm.at[b], idx_row)
            cps = [
                pltpu.async_copy(
                    emb_o_hbm.at[idx_row.at[c]], rows[c], gsem)
                for c in range(_NCH)
            ]
            for cp in cps:
                cp.wait()
            for c in range(_NCH):
                jvecs = [16 * g + lax.iota(jnp.int32, 16) for g in range(ngrp)]

                def kb_body(kb, accs, c=c, jvecs=jvecs):
                    accs = list(accs)
                    for k2 in range(16):
                        kvec = jnp.full((16,), kb * 16 + k2, jnp.int32)
                        ivk = plsc.load_gather(ivecs, [blvec, kvec])
                        accs = [
                            acc + plsc.load_gather(rows[c], [jv, kvec]) * ivk
                            for acc, jv in zip(accs, jvecs)
                        ]
                    return tuple(accs)

                accs = lax.fori_loop(
                    0, D // 16, kb_body,
                    tuple(jnp.zeros((16,), jnp.float32) for _ in range(ngrp)))
                for g in range(ngrp):
                    gi = c * _CH + 16 * g
                    lanes = gi + lax.iota(jnp.int32, 16)
                    sign = jnp.where(lanes < C, 1.0, -1.0).astype(jnp.float32)
                    scores_row[pl.ds(gi, 16)] = accs[g] * sign
            pltpu.sync_copy(scores_row, out_hbm.at[b])
            return carry

        lax.fori_loop(0, bpw, batch_body, 0)

    return body(iword, idx_all, emb_i, emb_o)


def _tc_reduce(scores, B, C, tot):
    blk = 512
    nblk = B // blk
    scale = 1.0 / (B * C)

    def body(x_ref, out_ref):
        i = pl.program_id(0)
        t = -x_ref[...]
        sp = jnp.maximum(t, 0.0) + jnp.log1p(jnp.exp(-jnp.abs(t)))
        col = lax.broadcasted_iota(jnp.int32, sp.shape, 1)
        part = jnp.sum(jnp.where(col < tot, sp, 0.0)) * scale

        @pl.when(i == 0)
        def _():
            out_ref[0, 0] = 0.0

        out_ref[0, 0] += part

    out = pl.pallas_call(
        body,
        grid=(nblk,),
        in_specs=[pl.BlockSpec((blk, _PAD), lambda i: (i, 0))],
        out_specs=pl.BlockSpec(memory_space=pltpu.SMEM),
        out_shape=jax.ShapeDtypeStruct((1, 1), jnp.float32),
    )(scores)
    return out[0, 0]


def kernel(iword, owords, emb_i, emb_o):
    B = iword.shape[0]
    C = owords.shape[1]
    V, D = emb_o.shape
    tot = C * (1 + _NNEG)
    nwords = jax.random.randint(jax.random.key(1234), (B, C * _NNEG), 0, V - 1)
    npad = _PAD - tot
    pad = (jnp.arange(npad, dtype=jnp.int32)[None, :] * 997
           + jnp.arange(B, dtype=jnp.int32)[:, None] * 31) % V
    idx_all = jnp.concatenate(
        [owords.astype(jnp.int32), nwords.astype(jnp.int32), pad],
        axis=1).reshape(B, _NCH, _CH)
    scores = _sc_scores(B, C, D, iword.astype(jnp.int32), idx_all,
                        emb_i, emb_o)
    return _tc_reduce(scores, B, C, tot)


# speedup vs baseline: 11.1997x; 11.1997x over previous
import functools

import jax
import jax.numpy as jnp
from jax import lax
from jax.experimental import pallas as pl
from jax.experimental.pallas import tpu as pltpu
from jax.experimental.pallas import tpu_sc as plsc

_NNEG = 20
_PAD = 448
_CH = 112
_NCH = 4
_NW = 32


def _sc_scores(B, C, D, iword, idx_all, emb_i, emb_o):
    bpw = B // _NW
    mesh = plsc.VectorSubcoreMesh(core_axis_name="c", subcore_axis_name="s")

    @functools.partial(
        pl.kernel,
        mesh=mesh,
        compiler_params=pltpu.CompilerParams(needs_layout_passes=False),
        out_type=jax.ShapeDtypeStruct((B, _PAD), jnp.float32),
        scratch_types=[
            pltpu.VMEM((bpw,), jnp.int32),
            pltpu.VMEM((8, D), jnp.float32),
        ] + [pltpu.VMEM((_NCH, _CH), jnp.int32) for _ in range(2)] + [
        ] + [pltpu.VMEM((_CH, D), jnp.float32) for _ in range(2 * _NCH)] + [
            pltpu.VMEM((_PAD,), jnp.float32),
            pltpu.VMEM((_PAD,), jnp.float32),
            pltpu.SemaphoreType.DMA,
            pltpu.SemaphoreType.DMA,
            pltpu.SemaphoreType.DMA,
            pltpu.SemaphoreType.DMA,
        ],
    )
    def body(iword_hbm, idx_hbm, emb_i_hbm, emb_o_hbm, out_hbm,
             iw_idx, ivecs, idx0, idx1, r00, r01, r02, r03, r10, r11, r12,
             r13, sc0, sc1, isem, gsem0, gsem1, ssem):
        idx_buf = [idx0, idx1]
        rows = [[r00, r01, r02, r03], [r10, r11, r12, r13]]
        scores = [sc0, sc1]
        wid = lax.axis_index("s") * 2 + lax.axis_index("c")
        base = wid * bpw
        pltpu.sync_copy(iword_hbm.at[pl.ds(base, bpw)], iw_idx)

        lane15 = lax.iota(jnp.int32, 16) == 15

        def issue_idx(bl, p):
            pltpu.async_copy(idx_hbm.at[base + bl], idx_buf[p], isem)

        def wait_idx():
            pltpu.make_async_copy(idx_hbm.at[base], idx_buf[0], isem).wait()

        def issue_gathers(p, gsem):
            for c in range(_NCH):
                pltpu.async_copy(
                    emb_o_hbm.at[idx_buf[p].at[c]], rows[p][c], gsem)

        def wait_gathers(p, gsem):
            for c in range(_NCH):
                pltpu.make_async_copy(
                    emb_o_hbm.at[pl.ds(0, _CH)], rows[p][c], gsem).wait()

        def wait_store():
            pltpu.make_async_copy(
                scores[0], out_hbm.at[base], ssem).wait()

        def compute(bl, p):
            @pl.when(bl % 8 == 0)
            def _():
                pltpu.async_copy(
                    emb_i_hbm.at[iw_idx.at[pl.ds(pl.multiple_of(bl, 8), 8)]],
                    ivecs, isem).wait()
            iv = [ivecs[bl % 8, pl.ds(16 * k, 16)] for k in range(D // 16)]
            for c in range(_NCH):
                @plsc.parallel_loop(0, _CH, unroll=8)
                def row_body(j, c=c):
                    prods = [rows[p][c][j, pl.ds(16 * k, 16)] * iv[k]
                             for k in range(D // 16)]
                    while len(prods) > 1:
                        prods = [a + b for a, b in zip(prods[::2], prods[1::2])]
                    cum = jnp.cumsum(prods[0])
                    plsc.store_scatter(
                        scores[p], [jnp.full((16,), c * _CH + j, jnp.int32)],
                        cum, mask=lane15)
            pltpu.async_copy(scores[p], out_hbm.at[base + bl], ssem)

        pltpu.sync_copy(idx_hbm.at[base], idx_buf[0])
        issue_gathers(0, gsem0)
        issue_idx(1, 1)

        def half(bl, p, gsem_self, gsem_next):
            nxt = bl + 1

            @pl.when(nxt < bpw)
            def _():
                wait_idx()
                issue_gathers(1 - p, gsem_next)
            wait_gathers(p, gsem_self)

            @pl.when(bl + 2 < bpw)
            def _():
                issue_idx(bl + 2, p)

            @pl.when(bl >= 2)
            def _():
                wait_store()
            compute(bl, p)

        def pair_body(t, carry):
            bl = 2 * t
            half(bl, 0, gsem0, gsem1)
            half(bl + 1, 1, gsem1, gsem0)
            return carry

        lax.fori_loop(0, bpw // 2, pair_body, 0)
        wait_store()
        wait_store()

    return body(iword, idx_all, emb_i, emb_o)


def _tc_reduce(scores, B, C, tot):
    blk = 512
    nblk = B // blk
    scale = 1.0 / (B * C)

    def body(x_ref, out_ref):
        i = pl.program_id(0)
        x = x_ref[...]
        col = lax.broadcasted_iota(jnp.int32, x.shape, 1)
        t = jnp.where(col < C, -x, x)
        sp = jnp.maximum(t, 0.0) + jnp.log1p(jnp.exp(-jnp.abs(t)))
        part = jnp.sum(jnp.where(col < tot, sp, 0.0)) * scale

        @pl.when(i == 0)
        def _():
            out_ref[0, 0] = 0.0

        out_ref[0, 0] += part

    out = pl.pallas_call(
        body,
        grid=(nblk,),
        in_specs=[pl.BlockSpec((blk, _PAD), lambda i: (i, 0))],
        out_specs=pl.BlockSpec(memory_space=pltpu.SMEM),
        out_shape=jax.ShapeDtypeStruct((1, 1), jnp.float32),
    )(scores)
    return out[0, 0]


def kernel(iword, owords, emb_i, emb_o):
    B = iword.shape[0]
    C = owords.shape[1]
    V, D = emb_o.shape
    tot = C * (1 + _NNEG)
    nwords = jax.random.randint(jax.random.key(1234), (B, C * _NNEG), 0, V - 1)
    npad = _PAD - tot
    pad = (jnp.arange(npad, dtype=jnp.int32)[None, :] * 997
           + jnp.arange(B, dtype=jnp.int32)[:, None] * 31) % V
    idx_all = jnp.concatenate(
        [owords.astype(jnp.int32), nwords.astype(jnp.int32), pad],
        axis=1).reshape(B, _NCH, _CH)
    scores = _sc_scores(B, C, D, iword.astype(jnp.int32), idx_all,
                        emb_i, emb_o)
    return _tc_reduce(scores, B, C, tot)
